# baseline (speedup 1.0000x reference)
import functools

import jax
import jax.numpy as jnp
from jax import lax
from jax.experimental import pallas as pl
from jax.experimental.pallas import tpu as pltpu
from jax.experimental.pallas import tpu_sc as plsc

NC = 2
NS = 16
NW = NC * NS
CHUNK = 128


def kernel(head, relation, tail, timestamp, entity_table, relation_table, timestamp_table):
    B = head.shape[0]
    NE, D = entity_table.shape
    b_per_w = B // NW
    nch = b_per_w // CHUNK
    R = CHUNK * D // CHUNK

    ent_flat = entity_table.T.reshape(NE * D)
    dvec = jnp.arange(D, dtype=jnp.int32) * NE
    wh = (head[:, None] + dvec[None, :]).reshape(NW, nch, R, CHUNK)
    wt = (tail[:, None] + dvec[None, :]).reshape(NW, nch, R, CHUNK)

    mesh = plsc.VectorSubcoreMesh(core_axis_name="c", subcore_axis_name="s")

    @functools.partial(
        pl.kernel,
        mesh=mesh,
        compiler_params=pltpu.CompilerParams(use_tc_tiling_on_sc=False),
        out_type=[
            jax.ShapeDtypeStruct((B * D // CHUNK, CHUNK), jnp.float32),
            jax.ShapeDtypeStruct((B, D), jnp.float32),
            jax.ShapeDtypeStruct((B * D // CHUNK, CHUNK), jnp.float32),
            jax.ShapeDtypeStruct((B, D), jnp.float32),
        ],
        scratch_types=[
            pltpu.VMEM((2 * nch, R, CHUNK), jnp.int32),
            pltpu.VMEM((2, nch, CHUNK), jnp.int32),
            pltpu.VMEM((2, R, CHUNK), jnp.float32),
            pltpu.VMEM((2, CHUNK, D), jnp.float32),
            pltpu.SemaphoreType.DMA,
            pltpu.SemaphoreType.DMA,
            pltpu.SemaphoreType.DMA,
            pltpu.SemaphoreType.DMA,
        ],
    )
    def gather4(wh_i, wt_i, r_i, ts_i, ent, rel, tst,
                out_h, out_r, out_t, out_ts,
                widx_v, sidx_v, wrows_v, rrows_v, gsem0, gsem1, ssem0, ssem1):
        gsems = [gsem0, gsem1]
        ssems = [ssem0, ssem1]
        wid = lax.axis_index("s") * NC + lax.axis_index("c")
        pltpu.sync_copy(wh_i.at[wid], widx_v.at[pl.ds(0, nch)])
        pltpu.sync_copy(wt_i.at[wid], widx_v.at[pl.ds(nch, nch)])
        pltpu.sync_copy(r_i.at[wid], sidx_v.at[0])
        pltpu.sync_copy(ts_i.at[wid], sidx_v.at[1])

        def fire(c, b):
            for j in range(R):
                pltpu.async_copy(ent.at[widx_v.at[c, j]], wrows_v.at[b, j], gsems[b])

        def drain(b):
            for j in range(R):
                pltpu.make_async_copy(
                    ent.at[widx_v.at[0, 0]], wrows_v.at[b, j], gsems[b]).wait()

        def copy_out(c, b):
            cc = c - (c >= nch) * nch
            rowbase = wid * (b_per_w * D // CHUNK) + cc * R

            @pl.when(c < nch)
            def _():
                pltpu.sync_copy(wrows_v.at[b], out_h.at[pl.ds(rowbase, R)])

            @pl.when(c >= nch)
            def _():
                pltpu.sync_copy(wrows_v.at[b], out_t.at[pl.ds(rowbase, R)])

        n_e = 2 * nch
        fire(0, 0)

        def ebody(c2, _):
            c = 2 * c2
            fire(c + 1, 1)
            drain(0)
            copy_out(c, 0)

            @pl.when(c + 2 < n_e)
            def _():
                fire(c + 2, 0)

            drain(1)
            copy_out(c + 1, 1)
            return _

        lax.fori_loop(0, n_e // 2, ebody, None)

        for t, (tab, out) in enumerate([(rel, out_r), (tst, out_ts)]):
            pltpu.async_copy(tab.at[sidx_v.at[t, 0]], rrows_v.at[0], ssems[0])
            for c in range(nch):
                b = c % 2
                if c + 1 < nch:
                    pltpu.async_copy(
                        tab.at[sidx_v.at[t, c + 1]], rrows_v.at[1 - b], ssems[1 - b])
                pltpu.make_async_copy(
                    tab.at[sidx_v.at[t, 0]], rrows_v.at[b], ssems[b]).wait()
                pltpu.sync_copy(rrows_v.at[b],
                                out.at[pl.ds(wid * b_per_w + c * CHUNK, CHUNK)])

    o_h, o_r, o_t, o_ts = gather4(
        wh, wt, relation.reshape(NW, nch, CHUNK), timestamp.reshape(NW, nch, CHUNK),
        ent_flat, relation_table, timestamp_table)
    return (o_h.reshape(B, D), o_r, o_t.reshape(B, D), o_ts)


# speedup vs baseline: 8.6215x; 8.6215x over previous
import functools

import jax
import jax.numpy as jnp
from jax import lax
from jax.experimental import pallas as pl
from jax.experimental.pallas import tpu as pltpu
from jax.experimental.pallas import tpu_sc as plsc

NC = 2
NS = 16
NW = NC * NS
CHUNK = 128
PD = 128


def kernel(head, relation, tail, timestamp, entity_table, relation_table, timestamp_table):
    B = head.shape[0]
    NE, D = entity_table.shape
    b_per_w = B // NW
    nch = b_per_w // CHUNK

    pad = lambda t: jnp.pad(t, ((0, 0), (0, PD - D)))
    ent_p = pad(entity_table)
    rel_p = pad(relation_table)
    ts_p = pad(timestamp_table)

    mesh = plsc.VectorSubcoreMesh(core_axis_name="c", subcore_axis_name="s")

    @functools.partial(
        pl.kernel,
        mesh=mesh,
        compiler_params=pltpu.CompilerParams(use_tc_tiling_on_sc=False),
        out_type=[jax.ShapeDtypeStruct((B, PD), jnp.float32)] * 4,
        scratch_types=[
            pltpu.VMEM((2 * nch, CHUNK), jnp.int32),
            pltpu.VMEM((2, nch, CHUNK), jnp.int32),
            pltpu.VMEM((2, CHUNK, PD), jnp.float32),
            pltpu.VMEM((2, CHUNK, PD), jnp.float32),
            pltpu.SemaphoreType.DMA,
            pltpu.SemaphoreType.DMA,
            pltpu.SemaphoreType.DMA,
            pltpu.SemaphoreType.DMA,
        ],
    )
    def gather4(h_i, t_i, r_i, ts_i, ent, rel, tst,
                out_h, out_t, out_r, out_ts,
                eidx_v, sidx_v, erows_v, rrows_v, gsem0, gsem1, ssem0, ssem1):
        gsems = [gsem0, gsem1]
        ssems = [ssem0, ssem1]
        wid = lax.axis_index("s") * NC + lax.axis_index("c")
        pltpu.sync_copy(h_i.at[wid], eidx_v.at[pl.ds(0, nch)])
        pltpu.sync_copy(t_i.at[wid], eidx_v.at[pl.ds(nch, nch)])
        pltpu.sync_copy(r_i.at[wid], sidx_v.at[0])
        pltpu.sync_copy(ts_i.at[wid], sidx_v.at[1])

        def fire(c, b):
            pltpu.async_copy(ent.at[eidx_v.at[c]], erows_v.at[b], gsems[b])

        def drain(b):
            pltpu.make_async_copy(
                ent.at[eidx_v.at[0]], erows_v.at[b], gsems[b]).wait()

        def copy_out(c, b):
            cc = c - (c >= nch) * nch
            base = wid * b_per_w + cc * CHUNK

            @pl.when(c < nch)
            def _():
                pltpu.sync_copy(erows_v.at[b], out_h.at[pl.ds(base, CHUNK)])

            @pl.when(c >= nch)
            def _():
                pltpu.sync_copy(erows_v.at[b], out_t.at[pl.ds(base, CHUNK)])

        n_e = 2 * nch
        fire(0, 0)

        def ebody(c2, _):
            c = 2 * c2
            fire(c + 1, 1)
            drain(0)
            copy_out(c, 0)

            @pl.when(c + 2 < n_e)
            def _():
                fire(c + 2, 0)

            drain(1)
            copy_out(c + 1, 1)
            return _

        lax.fori_loop(0, n_e // 2, ebody, None)

        for t, (tab, out) in enumerate([(rel, out_r), (tst, out_ts)]):
            pltpu.async_copy(tab.at[sidx_v.at[t, 0]], rrows_v.at[0], ssems[0])
            for c in range(nch):
                b = c % 2
                if c + 1 < nch:
                    pltpu.async_copy(
                        tab.at[sidx_v.at[t, c + 1]], rrows_v.at[1 - b], ssems[1 - b])
                pltpu.make_async_copy(
                    tab.at[sidx_v.at[t, 0]], rrows_v.at[b], ssems[b]).wait()
                pltpu.sync_copy(rrows_v.at[b],
                                out.at[pl.ds(wid * b_per_w + c * CHUNK, CHUNK)])

    o_h, o_t, o_r, o_ts = gather4(
        head.reshape(NW, nch, CHUNK), tail.reshape(NW, nch, CHUNK),
        relation.reshape(NW, nch, CHUNK), timestamp.reshape(NW, nch, CHUNK),
        ent_p, rel_p, ts_p)
    return (o_h[:, :D], o_r[:, :D], o_t[:, :D], o_ts[:, :D])


# speedup vs baseline: 8.6341x; 1.0015x over previous
import functools

import jax
import jax.numpy as jnp
from jax import lax
from jax.experimental import pallas as pl
from jax.experimental.pallas import tpu as pltpu
from jax.experimental.pallas import tpu_sc as plsc

NC = 2
NS = 16
NW = NC * NS
CHUNK = 128
PD = 128


def kernel(head, relation, tail, timestamp, entity_table, relation_table, timestamp_table):
    B = head.shape[0]
    NE, D = entity_table.shape
    b_per_w = B // NW
    nch = b_per_w // CHUNK

    pad = lambda t: jnp.pad(t, ((0, 0), (0, PD - D)))
    ent_p = pad(entity_table)
    rel_p = pad(relation_table)
    ts_p = pad(timestamp_table)

    eidx = jnp.concatenate(
        [head.reshape(NW, nch, CHUNK), tail.reshape(NW, nch, CHUNK)], axis=1)
    sidx = jnp.concatenate(
        [relation.reshape(NW, nch, CHUNK), timestamp.reshape(NW, nch, CHUNK)], axis=1)

    mesh = plsc.VectorSubcoreMesh(core_axis_name="c", subcore_axis_name="s")

    @functools.partial(
        pl.kernel,
        mesh=mesh,
        out_type=[jax.ShapeDtypeStruct((B, PD), jnp.float32)] * 4,
        scratch_types=[
            pltpu.VMEM((2 * nch, CHUNK), jnp.int32),
            pltpu.VMEM((2 * nch, CHUNK), jnp.int32),
            pltpu.VMEM((2, CHUNK, PD), jnp.float32),
            pltpu.VMEM((2, CHUNK, PD), jnp.float32),
            pltpu.SemaphoreType.DMA,
            pltpu.SemaphoreType.DMA,
            pltpu.SemaphoreType.DMA,
            pltpu.SemaphoreType.DMA,
        ],
    )
    def gather4(e_i, s_i, ent, rel, tst,
                out_h, out_t, out_r, out_ts,
                eidx_v, sidx_v, erows_v, rrows_v, gsem0, gsem1, ssem0, ssem1):
        gsems = [gsem0, gsem1]
        ssems = [ssem0, ssem1]
        wid = lax.axis_index("s") * NC + lax.axis_index("c")
        pltpu.sync_copy(e_i.at[wid], eidx_v)
        pltpu.sync_copy(s_i.at[wid], sidx_v)

        def fire(c, b):
            pltpu.async_copy(ent.at[eidx_v.at[c]], erows_v.at[b], gsems[b])

        def drain(b):
            pltpu.make_async_copy(
                ent.at[eidx_v.at[0]], erows_v.at[b], gsems[b]).wait()

        def copy_out(c, b):
            cc = c - (c >= nch) * nch
            base = wid * b_per_w + cc * CHUNK

            @pl.when(c < nch)
            def _():
                pltpu.sync_copy(erows_v.at[b], out_h.at[pl.ds(base, CHUNK)])

            @pl.when(c >= nch)
            def _():
                pltpu.sync_copy(erows_v.at[b], out_t.at[pl.ds(base, CHUNK)])

        n_e = 2 * nch
        fire(0, 0)

        def ebody(c2, _):
            c = 2 * c2
            fire(c + 1, 1)
            drain(0)
            copy_out(c, 0)

            @pl.when(c + 2 < n_e)
            def _():
                fire(c + 2, 0)

            drain(1)
            copy_out(c + 1, 1)
            return _

        lax.fori_loop(0, n_e // 2, ebody, None)

        for t, (tab, out) in enumerate([(rel, out_r), (tst, out_ts)]):
            pltpu.async_copy(tab.at[sidx_v.at[t * nch]], rrows_v.at[0], ssems[0])
            for c in range(nch):
                b = c % 2
                if c + 1 < nch:
                    pltpu.async_copy(
                        tab.at[sidx_v.at[t * nch + c + 1]], rrows_v.at[1 - b], ssems[1 - b])
                pltpu.make_async_copy(
                    tab.at[sidx_v.at[0]], rrows_v.at[b], ssems[b]).wait()
                pltpu.sync_copy(rrows_v.at[b],
                                out.at[pl.ds(wid * b_per_w + c * CHUNK, CHUNK)])

    o_h, o_t, o_r, o_ts = gather4(eidx, sidx, ent_p, rel_p, ts_p)
    return (o_h[:, :D], o_r[:, :D], o_t[:, :D], o_ts[:, :D])
